# Initial kernel scaffold; baseline (speedup 1.0000x reference)
import jax
import jax.numpy as jnp
from jax.experimental import pallas as pl


def kernel(x, z, mask, embed, Wi_f, Wh_f, b_f, Wi_b, Wh_b, b_b, Wk, Wv, q, Wo, Wout, bout):
    raise NotImplementedError("write your pallas kernel here")


# speedup vs baseline: 2.8145x; 2.8145x over previous
import functools

import jax
import jax.numpy as jnp
from jax import lax
from jax.experimental import pallas as pl
from jax.experimental.pallas import tpu as pltpu
from jax.experimental.pallas import tpu_sc as plsc

B, T, V, E, H = 16, 2048, 30000, 300, 200
ENC = 2 * H
NH = 4
HD = ENC // NH
G4 = 4 * H
TEST_TEMP = 0.001
K = int(round(20 / 100.0 * T))

EP = 512
EPW = EP // 2
CH = 128
NCH = T // CH
CH2 = 128
NC2 = T // CH2

BF = jnp.bfloat16
F32 = jnp.float32


_GCH = 128


def _sc_gather(table, idx):
    info = plsc.get_sparse_core_info()
    nc, ns = info.num_cores, info.num_subcores
    nw = nc * ns
    per_w = (B * T) // nw
    mesh = plsc.VectorSubcoreMesh(core_axis_name="c", subcore_axis_name="s")

    @functools.partial(
        pl.kernel,
        out_type=jax.ShapeDtypeStruct((B * T, EPW), jnp.int32),
        mesh=mesh,
        scratch_types=[
            pltpu.VMEM((_GCH,), jnp.int32),
            pltpu.VMEM((_GCH, EPW), jnp.int32),
            pltpu.SemaphoreType.DMA,
        ],
    )
    def k(table_hbm, idx_hbm, out_hbm, idx_v, rows_v, sem):
        wid = lax.axis_index("s") * nc + lax.axis_index("c")
        base = wid * per_w

        def body(i, _):
            off = base + i * _GCH
            pltpu.sync_copy(idx_hbm.at[pl.ds(off, _GCH)], idx_v)
            pltpu.async_copy(table_hbm.at[idx_v], rows_v, sem).wait()
            pltpu.sync_copy(rows_v, out_hbm.at[pl.ds(off, _GCH)])
            return 0

        lax.fori_loop(0, per_w // _GCH, body, 0)

    return k(table, idx)


def _scan_body(xf_ref, xb_ref, wif_ref, whf_ref, bf_ref,
               wib_ref, whb_ref, bb_ref, hfo_ref, hbo_ref,
               hf_s, cf_s, hb_s, cb_s):
    j = pl.program_id(0)

    @pl.when(j == 0)
    def _():
        hf_s[...] = jnp.zeros((B, H), BF)
        hb_s[...] = jnp.zeros((B, H), BF)
        cf_s[...] = jnp.zeros((B, H), F32)
        cb_s[...] = jnp.zeros((B, H), F32)

    wif = wif_ref[...]
    whf = whf_ref[...]
    wib = wib_ref[...]
    whb = whb_ref[...]
    bfv = bf_ref[...]
    bbv = bb_ref[...]

    def halfstep(xt, h, c, wi, wh, b):
        m1 = jnp.dot(xt, wi, preferred_element_type=F32)
        m2 = jnp.dot(h, wh, preferred_element_type=F32)
        g = (m1 + m2) + b
        i_ = jax.nn.sigmoid(g[:, :H])
        f_ = jax.nn.sigmoid(g[:, H:2 * H])
        gg = jnp.tanh(g[:, 2 * H:3 * H])
        o_ = jax.nn.sigmoid(g[:, 3 * H:])
        c = (f_ * c) + (i_ * gg)
        h = (o_ * jnp.tanh(c)).astype(BF)
        return h, c

    def step(s, carry):
        hf, cf, hb, cb = carry
        hf, cf = halfstep(xf_ref[s][:, :E], hf, cf, wif, whf, bfv)
        hfo_ref[s] = hf
        sb = CH - 1 - s
        hb, cb = halfstep(xb_ref[sb][:, :E], hb, cb, wib, whb, bbv)
        hbo_ref[sb] = hb
        return hf, cf, hb, cb

    carry = (hf_s[...], cf_s[...], hb_s[...], cb_s[...])
    hf, cf, hb, cb = lax.fori_loop(0, CH, step, carry)
    hf_s[...] = hf
    cf_s[...] = cf
    hb_s[...] = hb
    cb_s[...] = cb


def _scan(emb3, wif, whf, bfv, wib, whb, bbv):
    return pl.pallas_call(
        _scan_body,
        grid=(NCH,),
        in_specs=[
            pl.BlockSpec((CH, B, EP), lambda j: (j, 0, 0)),
            pl.BlockSpec((CH, B, EP), lambda j: (NCH - 1 - j, 0, 0)),
            pl.BlockSpec((E, G4), lambda j: (0, 0)),
            pl.BlockSpec((H, G4), lambda j: (0, 0)),
            pl.BlockSpec((1, G4), lambda j: (0, 0)),
            pl.BlockSpec((E, G4), lambda j: (0, 0)),
            pl.BlockSpec((H, G4), lambda j: (0, 0)),
            pl.BlockSpec((1, G4), lambda j: (0, 0)),
        ],
        out_specs=[
            pl.BlockSpec((CH, B, H), lambda j: (j, 0, 0)),
            pl.BlockSpec((CH, B, H), lambda j: (NCH - 1 - j, 0, 0)),
        ],
        out_shape=[
            jax.ShapeDtypeStruct((T, B, H), BF),
            jax.ShapeDtypeStruct((T, B, H), BF),
        ],
        scratch_shapes=[pltpu.VMEM((B, H), BF), pltpu.VMEM((B, H), F32),
                        pltpu.VMEM((B, H), BF), pltpu.VMEM((B, H), F32)],
    )(emb3, emb3, wif, whf, bfv, wib, whb, bbv)


def _kv_body(hf_ref, hb_ref, wk_ref, wv_ref, q_ref, vals_ref, lg_ref):
    n = CH2 * B
    hfr = hf_ref[...].reshape(n, H).astype(F32)
    hbr = hb_ref[...].reshape(n, H).astype(F32)
    hcat = jnp.concatenate([hfr, hbr], axis=1)
    keys = jnp.dot(hcat, wk_ref[...],
                   preferred_element_type=F32).astype(BF)
    vals_ref[...] = jnp.dot(hcat, wv_ref[...],
                            preferred_element_type=F32).astype(BF)
    kf = keys.astype(F32)
    qbf = q_ref[...].astype(BF).astype(F32)
    for h in range(NH):
        sl = slice(h * HD, (h + 1) * HD)
        lg = lax.dot_general(kf[:, sl], qbf[h:h + 1, :],
                             (((1,), (1,)), ((), ())),
                             preferred_element_type=F32)
        lg_ref[:, h:h + 1] = lg / TEST_TEMP


def _kv(hf, hb, Wk, Wv, q):
    return pl.pallas_call(
        _kv_body,
        grid=(NC2,),
        in_specs=[
            pl.BlockSpec((CH2, B, H), lambda i: (i, 0, 0)),
            pl.BlockSpec((CH2, B, H), lambda i: (i, 0, 0)),
            pl.BlockSpec((ENC, ENC), lambda i: (0, 0)),
            pl.BlockSpec((ENC, ENC), lambda i: (0, 0)),
            pl.BlockSpec((NH, HD), lambda i: (0, 0)),
        ],
        out_specs=[
            pl.BlockSpec((CH2 * B, ENC), lambda i: (i, 0)),
            pl.BlockSpec((CH2 * B, NH), lambda i: (i, 0)),
        ],
        out_shape=[
            jax.ShapeDtypeStruct((T * B, ENC), BF),
            jax.ShapeDtypeStruct((T * B, NH), F32),
        ],
    )(hf, hb, Wk, Wv, q)


def _attn_body(lg_ref, maskf_ref, attn_ref, zo_ref):
    lg = lg_ref[...].T

    minint = jnp.int32(-2147483648)
    maxpos = jnp.int32(2147483647)
    oi = lax.bitcast_convert_type(lg, jnp.int32)
    ordered = jnp.where(oi < 0, oi ^ maxpos, oi)

    def bit_step(i, p):
        c = p | jnp.left_shift(jnp.int32(1), 31 - i)
        sc = c ^ minint
        cnt = jnp.sum((ordered >= sc).astype(jnp.int32), axis=1, keepdims=True)
        return jnp.where(cnt >= K, c, p)

    p = lax.fori_loop(0, 32, bit_step, jnp.zeros((B * NH, 1), jnp.int32))
    thr = p ^ minint
    sel = ordered >= thr

    masked = jnp.where(sel, lg, jnp.float32(-1e30))
    m = jnp.max(masked, axis=1, keepdims=True)
    e = jnp.exp(masked - m)
    z = jnp.sum(e, axis=1, keepdims=True)
    attn = e / z
    attn_ref[...] = attn

    a3 = attn.reshape(B, NH, T)
    zs = ((a3[:, 0, :] + a3[:, 1, :]) + a3[:, 2, :]) + a3[:, 3, :]
    zo_ref[...] = (zs * (1.0 / NH)) * maskf_ref[...]


def _attn(lg2, maskf):
    return pl.pallas_call(
        _attn_body,
        out_shape=[
            jax.ShapeDtypeStruct((B * NH, T), F32),
            jax.ShapeDtypeStruct((B, T), F32),
        ],
    )(lg2, maskf)


BG = 8


def _tail_body(attn_ref, vals_ref, wo_ref, wout_ref, bout_ref, y_ref,
               pooled_s):
    attn_bf = attn_ref[...].astype(BF)
    for b in range(BG):
        vb = vals_ref[:, b, :]
        full = jnp.dot(attn_bf[NH * b:NH * (b + 1), :], vb,
                       preferred_element_type=F32)
        for h in range(NH):
            sl = slice(h * HD, (h + 1) * HD)
            pooled_s[b:b + 1, sl] = full[h:h + 1, sl]
    pooled = pooled_s[...]
    p_bf = pooled.astype(BF).astype(F32)
    p2 = jnp.dot(p_bf, wo_ref[...], preferred_element_type=F32)
    p2_bf = p2.astype(BF).astype(F32)
    s = jnp.dot(p2_bf, wout_ref[...], preferred_element_type=F32)
    y_ref[...] = jax.nn.sigmoid(s + bout_ref[...])


def _tail(attn, vals3, Wo, Wout, boutv):
    return pl.pallas_call(
        _tail_body,
        grid=(B // BG,),
        in_specs=[
            pl.BlockSpec((NH * BG, T), lambda i: (i, 0)),
            pl.BlockSpec((T, BG, ENC), lambda i: (0, i, 0)),
            pl.BlockSpec((ENC, ENC), lambda i: (0, 0)),
            pl.BlockSpec((ENC, 1), lambda i: (0, 0)),
            pl.BlockSpec((1, 1), lambda i: (0, 0)),
        ],
        out_specs=pl.BlockSpec((BG, 1), lambda i: (i, 0)),
        out_shape=jax.ShapeDtypeStruct((B, 1), F32),
        scratch_shapes=[pltpu.VMEM((BG, ENC), F32)],
    )(attn, vals3, Wo, Wout, boutv)


def kernel(x, z, mask, embed, Wi_f, Wh_f, b_f, Wi_b, Wh_b, b_b,
           Wk, Wv, q, Wo, Wout, bout):
    x_flat = x.astype(jnp.int32).T.reshape(B * T)
    embed_bf = jnp.pad(embed.astype(BF), ((0, 0), (0, EP - E)))
    embed_i32 = lax.bitcast_convert_type(
        embed_bf.reshape(V, EPW, 2), jnp.int32)
    maskf = mask.astype(F32)
    boutv = bout.reshape(1, 1)

    emb_i32 = _sc_gather(embed_i32, x_flat)
    emb = lax.bitcast_convert_type(emb_i32.reshape(B * T, EPW, 1),
                                   BF).reshape(B * T, EP)
    hf, hb = _scan(emb.reshape(T, B, EP),
                   Wi_f.astype(BF), Wh_f.astype(BF), b_f.reshape(1, G4),
                   Wi_b.astype(BF), Wh_b.astype(BF), b_b.reshape(1, G4))
    vals, lg = _kv(hf, hb, Wk, Wv, q)
    attn, zo = _attn(lg.reshape(T, B * NH), maskf)
    y = _tail(attn, vals.reshape(T, B, ENC), Wo, Wout, boutv)
    return y, zo


# speedup vs baseline: 3.0456x; 1.0821x over previous
import functools

import jax
import jax.numpy as jnp
from jax import lax
from jax.experimental import pallas as pl
from jax.experimental.pallas import tpu as pltpu
from jax.experimental.pallas import tpu_sc as plsc

B, T, V, E, H = 16, 2048, 30000, 300, 200
ENC = 2 * H
NH = 4
HD = ENC // NH
G4 = 4 * H
TEST_TEMP = 0.001
K = int(round(20 / 100.0 * T))

EP = 512
EPW = EP // 2
CH = 128
NCH = T // CH
CH2 = 128
NC2 = T // CH2

BF = jnp.bfloat16
F32 = jnp.float32


_GCH = 128


def _sc_gather(table, idx):
    info = plsc.get_sparse_core_info()
    nc, ns = info.num_cores, info.num_subcores
    nw = nc * ns
    per_w = (B * T) // nw
    mesh = plsc.VectorSubcoreMesh(core_axis_name="c", subcore_axis_name="s")

    @functools.partial(
        pl.kernel,
        out_type=jax.ShapeDtypeStruct((B * T, EPW), jnp.int32),
        mesh=mesh,
        scratch_types=[
            pltpu.VMEM((_GCH,), jnp.int32),
            pltpu.VMEM((_GCH, EPW), jnp.int32),
            pltpu.SemaphoreType.DMA,
        ],
    )
    def k(table_hbm, idx_hbm, out_hbm, idx_v, rows_v, sem):
        wid = lax.axis_index("s") * nc + lax.axis_index("c")
        base = wid * per_w

        def body(i, _):
            off = base + i * _GCH
            pltpu.sync_copy(idx_hbm.at[pl.ds(off, _GCH)], idx_v)
            pltpu.async_copy(table_hbm.at[idx_v], rows_v, sem).wait()
            pltpu.sync_copy(rows_v, out_hbm.at[pl.ds(off, _GCH)])
            return 0

        lax.fori_loop(0, per_w // _GCH, body, 0)

    return k(table, idx)


PRB = 64
NPRB = T // PRB


def _proj_body(emb_ref, wif_ref, wib_ref, gf_ref, gb_ref):
    xt = emb_ref[...][:, :, :E].reshape(PRB * B, E)
    gf_ref[...] = jnp.dot(xt, wif_ref[...],
                          preferred_element_type=F32).reshape(PRB, B, G4)
    gb_ref[...] = jnp.dot(xt, wib_ref[...],
                          preferred_element_type=F32).reshape(PRB, B, G4)


def _proj(emb3, wif, wib):
    return pl.pallas_call(
        _proj_body,
        grid=(NPRB,),
        in_specs=[
            pl.BlockSpec((PRB, B, EP), lambda i: (i, 0, 0)),
            pl.BlockSpec((E, G4), lambda i: (0, 0)),
            pl.BlockSpec((E, G4), lambda i: (0, 0)),
        ],
        out_specs=[
            pl.BlockSpec((PRB, B, G4), lambda i: (i, 0, 0)),
            pl.BlockSpec((PRB, B, G4), lambda i: (i, 0, 0)),
        ],
        out_shape=[
            jax.ShapeDtypeStruct((T, B, G4), F32),
            jax.ShapeDtypeStruct((T, B, G4), F32),
        ],
    )(emb3, wif, wib)


def _scan_body(xf_ref, xb_ref, whf_ref, bf_ref,
               whb_ref, bb_ref, hfo_ref, hbo_ref,
               hf_s, cf_s, hb_s, cb_s):
    j = pl.program_id(0)

    @pl.when(j == 0)
    def _():
        hf_s[...] = jnp.zeros((B, H), BF)
        hb_s[...] = jnp.zeros((B, H), BF)
        cf_s[...] = jnp.zeros((B, H), F32)
        cb_s[...] = jnp.zeros((B, H), F32)

    whf = whf_ref[...]
    whb = whb_ref[...]
    bfv = bf_ref[...]
    bbv = bb_ref[...]

    def halfstep(m1, h, c, wh, b):
        m2 = jnp.dot(h, wh, preferred_element_type=F32)
        g = (m1 + m2) + b
        i_ = jax.nn.sigmoid(g[:, :H])
        f_ = jax.nn.sigmoid(g[:, H:2 * H])
        gg = jnp.tanh(g[:, 2 * H:3 * H])
        o_ = jax.nn.sigmoid(g[:, 3 * H:])
        c = (f_ * c) + (i_ * gg)
        h = (o_ * jnp.tanh(c)).astype(BF)
        return h, c

    def step(s, carry):
        hf, cf, hb, cb = carry
        hf, cf = halfstep(xf_ref[s], hf, cf, whf, bfv)
        hfo_ref[s] = hf
        sb = CH - 1 - s
        hb, cb = halfstep(xb_ref[sb], hb, cb, whb, bbv)
        hbo_ref[sb] = hb
        return hf, cf, hb, cb

    carry = (hf_s[...], cf_s[...], hb_s[...], cb_s[...])
    hf, cf, hb, cb = lax.fori_loop(0, CH, step, carry)
    hf_s[...] = hf
    cf_s[...] = cf
    hb_s[...] = hb
    cb_s[...] = cb


def _scan(gf, gb, whf, bfv, whb, bbv):
    return pl.pallas_call(
        _scan_body,
        grid=(NCH,),
        in_specs=[
            pl.BlockSpec((CH, B, G4), lambda j: (j, 0, 0)),
            pl.BlockSpec((CH, B, G4), lambda j: (NCH - 1 - j, 0, 0)),
            pl.BlockSpec((H, G4), lambda j: (0, 0)),
            pl.BlockSpec((1, G4), lambda j: (0, 0)),
            pl.BlockSpec((H, G4), lambda j: (0, 0)),
            pl.BlockSpec((1, G4), lambda j: (0, 0)),
        ],
        out_specs=[
            pl.BlockSpec((CH, B, H), lambda j: (j, 0, 0)),
            pl.BlockSpec((CH, B, H), lambda j: (NCH - 1 - j, 0, 0)),
        ],
        out_shape=[
            jax.ShapeDtypeStruct((T, B, H), BF),
            jax.ShapeDtypeStruct((T, B, H), BF),
        ],
        scratch_shapes=[pltpu.VMEM((B, H), BF), pltpu.VMEM((B, H), F32),
                        pltpu.VMEM((B, H), BF), pltpu.VMEM((B, H), F32)],
    )(gf, gb, whf, bfv, whb, bbv)


def _kv_body(hf_ref, hb_ref, wk_ref, wv_ref, q_ref, vals_ref, lg_ref):
    n = CH2 * B
    hfr = hf_ref[...].reshape(n, H).astype(F32)
    hbr = hb_ref[...].reshape(n, H).astype(F32)
    hcat = jnp.concatenate([hfr, hbr], axis=1)
    keys = jnp.dot(hcat, wk_ref[...],
                   preferred_element_type=F32).astype(BF)
    vals_ref[...] = jnp.dot(hcat, wv_ref[...],
                            preferred_element_type=F32).astype(BF)
    kf = keys.astype(F32)
    qbf = q_ref[...].astype(BF).astype(F32)
    for h in range(NH):
        sl = slice(h * HD, (h + 1) * HD)
        lg = lax.dot_general(kf[:, sl], qbf[h:h + 1, :],
                             (((1,), (1,)), ((), ())),
                             preferred_element_type=F32)
        lg_ref[:, h:h + 1] = lg / TEST_TEMP


def _kv(hf, hb, Wk, Wv, q):
    return pl.pallas_call(
        _kv_body,
        grid=(NC2,),
        in_specs=[
            pl.BlockSpec((CH2, B, H), lambda i: (i, 0, 0)),
            pl.BlockSpec((CH2, B, H), lambda i: (i, 0, 0)),
            pl.BlockSpec((ENC, ENC), lambda i: (0, 0)),
            pl.BlockSpec((ENC, ENC), lambda i: (0, 0)),
            pl.BlockSpec((NH, HD), lambda i: (0, 0)),
        ],
        out_specs=[
            pl.BlockSpec((CH2 * B, ENC), lambda i: (i, 0)),
            pl.BlockSpec((CH2 * B, NH), lambda i: (i, 0)),
        ],
        out_shape=[
            jax.ShapeDtypeStruct((T * B, ENC), BF),
            jax.ShapeDtypeStruct((T * B, NH), F32),
        ],
    )(hf, hb, Wk, Wv, q)


def _attn_body(lg_ref, maskf_ref, attn_ref, zo_ref):
    lg = lg_ref[...].T

    minint = jnp.int32(-2147483648)
    maxpos = jnp.int32(2147483647)
    oi = lax.bitcast_convert_type(lg, jnp.int32)
    ordered = jnp.where(oi < 0, oi ^ maxpos, oi)

    def bit_step(i, p):
        c = p | jnp.left_shift(jnp.int32(1), 31 - i)
        sc = c ^ minint
        cnt = jnp.sum((ordered >= sc).astype(jnp.int32), axis=1, keepdims=True)
        return jnp.where(cnt >= K, c, p)

    p = lax.fori_loop(0, 32, bit_step, jnp.zeros((B * NH, 1), jnp.int32))
    thr = p ^ minint
    sel = ordered >= thr

    masked = jnp.where(sel, lg, jnp.float32(-1e30))
    m = jnp.max(masked, axis=1, keepdims=True)
    e = jnp.exp(masked - m)
    z = jnp.sum(e, axis=1, keepdims=True)
    attn = e / z
    attn_ref[...] = attn

    a3 = attn.reshape(B, NH, T)
    zs = ((a3[:, 0, :] + a3[:, 1, :]) + a3[:, 2, :]) + a3[:, 3, :]
    zo_ref[...] = (zs * (1.0 / NH)) * maskf_ref[...]


def _attn(lg2, maskf):
    return pl.pallas_call(
        _attn_body,
        out_shape=[
            jax.ShapeDtypeStruct((B * NH, T), F32),
            jax.ShapeDtypeStruct((B, T), F32),
        ],
    )(lg2, maskf)


BG = 8


def _tail_body(attn_ref, vals_ref, wo_ref, wout_ref, bout_ref, y_ref,
               pooled_s):
    attn_bf = attn_ref[...].astype(BF)
    for b in range(BG):
        vb = vals_ref[:, b, :]
        full = jnp.dot(attn_bf[NH * b:NH * (b + 1), :], vb,
                       preferred_element_type=F32)
        for h in range(NH):
            sl = slice(h * HD, (h + 1) * HD)
            pooled_s[b:b + 1, sl] = full[h:h + 1, sl]
    pooled = pooled_s[...]
    p_bf = pooled.astype(BF).astype(F32)
    p2 = jnp.dot(p_bf, wo_ref[...], preferred_element_type=F32)
    p2_bf = p2.astype(BF).astype(F32)
    s = jnp.dot(p2_bf, wout_ref[...], preferred_element_type=F32)
    y_ref[...] = jax.nn.sigmoid(s + bout_ref[...])


def _tail(attn, vals3, Wo, Wout, boutv):
    return pl.pallas_call(
        _tail_body,
        grid=(B // BG,),
        in_specs=[
            pl.BlockSpec((NH * BG, T), lambda i: (i, 0)),
            pl.BlockSpec((T, BG, ENC), lambda i: (0, i, 0)),
            pl.BlockSpec((ENC, ENC), lambda i: (0, 0)),
            pl.BlockSpec((ENC, 1), lambda i: (0, 0)),
            pl.BlockSpec((1, 1), lambda i: (0, 0)),
        ],
        out_specs=pl.BlockSpec((BG, 1), lambda i: (i, 0)),
        out_shape=jax.ShapeDtypeStruct((B, 1), F32),
        scratch_shapes=[pltpu.VMEM((BG, ENC), F32)],
    )(attn, vals3, Wo, Wout, boutv)


def kernel(x, z, mask, embed, Wi_f, Wh_f, b_f, Wi_b, Wh_b, b_b,
           Wk, Wv, q, Wo, Wout, bout):
    x_flat = x.astype(jnp.int32).T.reshape(B * T)
    embed_bf = jnp.pad(embed.astype(BF), ((0, 0), (0, EP - E)))
    embed_i32 = lax.bitcast_convert_type(
        embed_bf.reshape(V, EPW, 2), jnp.int32)
    maskf = mask.astype(F32)
    boutv = bout.reshape(1, 1)

    emb_i32 = _sc_gather(embed_i32, x_flat)
    emb = lax.bitcast_convert_type(emb_i32.reshape(B * T, EPW, 1),
                                   BF).reshape(B * T, EP)
    gf, gb = _proj(emb.reshape(T, B, EP),
                   Wi_f.astype(BF), Wi_b.astype(BF))
    hf, hb = _scan(gf, gb, Wh_f.astype(BF), b_f.reshape(1, G4),
                   Wh_b.astype(BF), b_b.reshape(1, G4))
    vals, lg = _kv(hf, hb, Wk, Wv, q)
    attn, zo = _attn(lg.reshape(T, B * NH), maskf)
    y = _tail(attn, vals.reshape(T, B, ENC), Wo, Wout, boutv)
    return y, zo
